# Initial kernel scaffold; baseline (speedup 1.0000x reference)
import functools

import jax
import jax.numpy as jnp
from jax import lax
from jax.experimental import pallas as pl
from jax.experimental.pallas import tpu as pltpu
from jax.experimental.pallas import tpu_sc as plsc

N_ROWS = 100000
D = 128
NSEG = 1024
NC = 2
NS = 16
NW = NC * NS
B = 128
FULL_BLOCKS = N_ROWS // B
TAIL = N_ROWS - FULL_BLOCKS * B
ITERS = (FULL_BLOCKS + NW - 1) // NW
SEG_PER_TILE = NSEG // NS
CW = 16


def _sc_partials(node_feat, seg_ids, ones_rows, zsum, zcnt):
    mesh = plsc.VectorSubcoreMesh(core_axis_name="c", subcore_axis_name="s")

    @functools.partial(
        pl.kernel,
        out_type=(
            jax.ShapeDtypeStruct((NC, NSEG, D), jnp.float32),
            jax.ShapeDtypeStruct((NC, NSEG, CW), jnp.float32),
        ),
        mesh=mesh,
        scratch_types=[
            pltpu.VMEM((B,), jnp.int32),
            pltpu.VMEM((B, D), jnp.float32),
            pltpu.VMEM((B, CW), jnp.float32),
            pltpu.VMEM((SEG_PER_TILE, D), jnp.float32),
            pltpu.VMEM((SEG_PER_TILE, CW), jnp.float32),
            pltpu.VMEM((TAIL,), jnp.int32),
            pltpu.VMEM((TAIL, D), jnp.float32),
            pltpu.VMEM_SHARED((NSEG, D), jnp.float32),
            pltpu.VMEM_SHARED((NSEG, CW), jnp.float32),
            pltpu.SemaphoreType.DMA,
            pltpu.SemaphoreType.DMA,
        ],
    )
    def k(feat_hbm, ids_hbm, ones_hbm, zsum_hbm, zcnt_hbm,
          psum_hbm, pcnt_hbm,
          idx_v, rows_v, ones_v, zer_v, zcnt_v, idx_t, rows_t,
          sum_sh, cnt_sh, sem1, sem2):
        c = lax.axis_index("c")
        s = lax.axis_index("s")
        wid = c * NS + s

        pltpu.sync_copy(ones_hbm, ones_v)
        pltpu.sync_copy(zsum_hbm, zer_v)
        pltpu.sync_copy(zcnt_hbm, zcnt_v)
        pltpu.sync_copy(zer_v, sum_sh.at[pl.ds(s * SEG_PER_TILE, SEG_PER_TILE)])
        pltpu.sync_copy(zcnt_v, cnt_sh.at[pl.ds(s * SEG_PER_TILE, SEG_PER_TILE)])
        plsc.subcore_barrier()

        def block_body(i, carry):
            b = wid + i * NW

            @pl.when(b < FULL_BLOCKS)
            def _():
                base = b * B
                pltpu.sync_copy(ids_hbm.at[pl.ds(base, B)], idx_v)
                pltpu.sync_copy(feat_hbm.at[pl.ds(base, B), :], rows_v)
                d1 = pltpu.async_copy(rows_v, sum_sh.at[idx_v], sem1, add=True)
                d2 = pltpu.async_copy(ones_v, cnt_sh.at[idx_v], sem2, add=True)
                d1.wait()
                d2.wait()

            return carry

        lax.fori_loop(0, ITERS, block_body, 0)

        @pl.when(wid == NW - 1)
        def _():
            base = FULL_BLOCKS * B
            pltpu.sync_copy(ids_hbm.at[pl.ds(base, TAIL)], idx_t)
            pltpu.sync_copy(feat_hbm.at[pl.ds(base, TAIL), :], rows_t)
            d1 = pltpu.async_copy(rows_t, sum_sh.at[idx_t], sem1, add=True)
            d2 = pltpu.async_copy(ones_hbm.at[pl.ds(0, TAIL), :],
                                  cnt_sh.at[idx_t], sem2, add=True)
            d1.wait()
            d2.wait()

        plsc.subcore_barrier()

        seg0 = s * SEG_PER_TILE
        pltpu.sync_copy(sum_sh.at[pl.ds(seg0, SEG_PER_TILE)], zer_v)
        pltpu.sync_copy(cnt_sh.at[pl.ds(seg0, SEG_PER_TILE)], zcnt_v)
        pltpu.sync_copy(zer_v, psum_hbm.at[c, pl.ds(seg0, SEG_PER_TILE), :])
        pltpu.sync_copy(zcnt_v, pcnt_hbm.at[c, pl.ds(seg0, SEG_PER_TILE), :])

    return k(node_feat, seg_ids, ones_rows, zsum, zcnt)


def _combine_body(psum_ref, pcnt_ref, out_ref):
    sums = psum_ref[0] + psum_ref[1]
    cnts = pcnt_ref[0, :, 0:1] + pcnt_ref[1, :, 0:1]
    out_ref[...] = sums / jnp.maximum(cnts, 1.0)


def kernel(node_feat, segment_ids):
    ids32 = segment_ids.astype(jnp.int32)
    ones_rows = jnp.ones((B, CW), jnp.float32)
    zsum = jnp.zeros((SEG_PER_TILE, D), jnp.float32)
    zcnt = jnp.zeros((SEG_PER_TILE, CW), jnp.float32)
    psum, pcnt = _sc_partials(node_feat, ids32, ones_rows, zsum, zcnt)
    return pl.pallas_call(
        _combine_body,
        out_shape=jax.ShapeDtypeStruct((NSEG, D), jnp.float32),
    )(psum, pcnt)


# speedup vs baseline: 4.5793x; 4.5793x over previous
import functools

import jax
import jax.numpy as jnp
from jax import lax
from jax.experimental import pallas as pl
from jax.experimental.pallas import tpu as pltpu
from jax.experimental.pallas import tpu_sc as plsc

N_ROWS = 100000
D = 128
NSEG = 1024
NC = 2
NS = 16
NW = NC * NS
B = 128
FULL_BLOCKS = N_ROWS // B
TAIL = N_ROWS - FULL_BLOCKS * B
ITERS = (FULL_BLOCKS + NW - 1) // NW
SEG_PER_TILE = NSEG // NS
CW = 128


def _sc_partials(node_feat, seg_ids, ones_rows, zsum, zcnt):
    mesh = plsc.VectorSubcoreMesh(core_axis_name="c", subcore_axis_name="s",
                                  num_cores=NC, num_subcores=NS)

    @functools.partial(
        pl.kernel,
        out_type=(
            jax.ShapeDtypeStruct((NC, NSEG, D), jnp.float32),
            jax.ShapeDtypeStruct((NC, NSEG, CW), jnp.float32),
        ),
        mesh=mesh,
        scratch_types=[
            pltpu.VMEM((B,), jnp.int32),
            pltpu.VMEM((B, D), jnp.float32),
            pltpu.VMEM((B, CW), jnp.float32),
            pltpu.VMEM((SEG_PER_TILE, D), jnp.float32),
            pltpu.VMEM((SEG_PER_TILE, CW), jnp.float32),
            pltpu.VMEM((TAIL,), jnp.int32),
            pltpu.VMEM((TAIL, D), jnp.float32),
            pltpu.VMEM((TAIL, CW), jnp.float32),
            pltpu.VMEM_SHARED((NSEG, D), jnp.float32),
            pltpu.VMEM_SHARED((NSEG, CW), jnp.float32),
            pltpu.SemaphoreType.DMA,
            pltpu.SemaphoreType.DMA,
        ],
    )
    def k(feat_hbm, ids_hbm, ones_hbm, zsum_hbm, zcnt_hbm,
          psum_hbm, pcnt_hbm,
          idx_v, rows_v, ones_v, zer_v, zcnt_v, idx_t, rows_t, ones_t,
          sum_sh, cnt_sh, sem1, sem2):
        c = lax.axis_index("c")
        s = lax.axis_index("s")
        wid = c * NS + s

        pltpu.sync_copy(ones_hbm, ones_v)
        pltpu.sync_copy(zsum_hbm, zer_v)
        pltpu.sync_copy(zcnt_hbm, zcnt_v)
        pltpu.sync_copy(zer_v, sum_sh.at[pl.ds(s * SEG_PER_TILE, SEG_PER_TILE)])
        pltpu.sync_copy(zcnt_v, cnt_sh.at[pl.ds(s * SEG_PER_TILE, SEG_PER_TILE)])
        plsc.subcore_barrier()

        def block_body(i, carry):
            b = wid + i * NW

            @pl.when(b < FULL_BLOCKS)
            def _():
                base = b * B
                pltpu.sync_copy(ids_hbm.at[pl.ds(base, B)], idx_v)
                pltpu.sync_copy(feat_hbm.at[pl.ds(base, B), :], rows_v)
                d1 = pltpu.async_copy(rows_v, sum_sh.at[idx_v], sem1, add=True)
                d2 = pltpu.async_copy(ones_v, cnt_sh.at[idx_v], sem2, add=True)
                d1.wait()
                d2.wait()

            return carry

        lax.fori_loop(0, ITERS, block_body, 0)

        @pl.when(wid == NW - 1)
        def _():
            base = FULL_BLOCKS * B
            pltpu.sync_copy(ids_hbm.at[pl.ds(base, TAIL)], idx_t)
            pltpu.sync_copy(feat_hbm.at[pl.ds(base, TAIL), :], rows_t)
            pltpu.sync_copy(ones_hbm.at[pl.ds(0, TAIL), :], ones_t)
            d1 = pltpu.async_copy(rows_t, sum_sh.at[idx_t], sem1, add=True)
            d2 = pltpu.async_copy(ones_t, cnt_sh.at[idx_t], sem2, add=True)
            d1.wait()
            d2.wait()

        plsc.subcore_barrier()

        seg0 = s * SEG_PER_TILE
        pltpu.sync_copy(sum_sh.at[pl.ds(seg0, SEG_PER_TILE)], zer_v)
        pltpu.sync_copy(cnt_sh.at[pl.ds(seg0, SEG_PER_TILE)], zcnt_v)
        pltpu.sync_copy(zer_v, psum_hbm.at[c, pl.ds(seg0, SEG_PER_TILE), :])
        pltpu.sync_copy(zcnt_v, pcnt_hbm.at[c, pl.ds(seg0, SEG_PER_TILE), :])

    return k(node_feat, seg_ids, ones_rows, zsum, zcnt)


def _combine_body(psum_ref, pcnt_ref, out_ref):
    sums = psum_ref[0] + psum_ref[1]
    cnts = pcnt_ref[0, :, 0:1] + pcnt_ref[1, :, 0:1]
    out_ref[...] = sums / jnp.maximum(cnts, 1.0)


def kernel(node_feat, segment_ids):
    ids32 = segment_ids.astype(jnp.int32)
    ones_rows = jnp.ones((B, CW), jnp.float32)
    zsum = jnp.zeros((SEG_PER_TILE, D), jnp.float32)
    zcnt = jnp.zeros((SEG_PER_TILE, CW), jnp.float32)
    psum, pcnt = _sc_partials(node_feat, ids32, ones_rows, zsum, zcnt)
    return pl.pallas_call(
        _combine_body,
        out_shape=jax.ShapeDtypeStruct((NSEG, D), jnp.float32),
    )(psum, pcnt)


# speedup vs baseline: 6.3444x; 1.3855x over previous
import functools

import jax
import jax.numpy as jnp
from jax import lax
from jax.experimental import pallas as pl
from jax.experimental.pallas import tpu as pltpu
from jax.experimental.pallas import tpu_sc as plsc

N_ROWS = 100000
D = 128
NSEG = 1024
NC = 2
NS = 16
NW = NC * NS
B = 128
FULL_BLOCKS = N_ROWS // B
TAIL = N_ROWS - FULL_BLOCKS * B
ITERS = (FULL_BLOCKS + NW - 1) // NW
SEG_PER_TILE = NSEG // NS
CW = 128


def _sc_partials(node_feat, seg_ids, ones_rows, zsum, zcnt):
    mesh = plsc.VectorSubcoreMesh(core_axis_name="c", subcore_axis_name="s",
                                  num_cores=NC, num_subcores=NS)

    @functools.partial(
        pl.kernel,
        out_type=(
            jax.ShapeDtypeStruct((NC, NSEG, D), jnp.float32),
            jax.ShapeDtypeStruct((NC, NSEG, CW), jnp.float32),
        ),
        mesh=mesh,
        scratch_types=[
            pltpu.VMEM((B,), jnp.int32),
            pltpu.VMEM((B,), jnp.int32),
            pltpu.VMEM((B, D), jnp.float32),
            pltpu.VMEM((B, D), jnp.float32),
            pltpu.VMEM((B, CW), jnp.float32),
            pltpu.VMEM((SEG_PER_TILE, D), jnp.float32),
            pltpu.VMEM((SEG_PER_TILE, CW), jnp.float32),
            pltpu.VMEM((TAIL,), jnp.int32),
            pltpu.VMEM((TAIL, D), jnp.float32),
            pltpu.VMEM((TAIL, CW), jnp.float32),
            pltpu.VMEM_SHARED((NSEG, D), jnp.float32),
            pltpu.VMEM_SHARED((NSEG, CW), jnp.float32),
            [pltpu.SemaphoreType.DMA] * 8,
        ],
    )
    def k(feat_hbm, ids_hbm, ones_hbm, zsum_hbm, zcnt_hbm,
          psum_hbm, pcnt_hbm,
          idx0_v, idx1_v, rows0_v, rows1_v, ones_v, zer_v, zcnt_v,
          idx_t, rows_t, ones_t,
          sum_sh, cnt_sh, sems):
        (sem_gi0, sem_gr0, sem_gi1, sem_gr1,
         sem_s0a, sem_s0b, sem_s1a, sem_s1b) = sems
        c = lax.axis_index("c")
        s = lax.axis_index("s")
        wid = c * NS + s

        pltpu.sync_copy(ones_hbm, ones_v)
        pltpu.sync_copy(zsum_hbm, zer_v)
        pltpu.sync_copy(zcnt_hbm, zcnt_v)
        pltpu.sync_copy(zer_v, sum_sh.at[pl.ds(s * SEG_PER_TILE, SEG_PER_TILE)])
        pltpu.sync_copy(zcnt_v, cnt_sh.at[pl.ds(s * SEG_PER_TILE, SEG_PER_TILE)])
        plsc.subcore_barrier()

        def start_gather(b, idx_v, rows_v, sem_i, sem_r):
            base = b * B
            pltpu.async_copy(ids_hbm.at[pl.ds(base, B)], idx_v, sem_i)
            pltpu.async_copy(feat_hbm.at[pl.ds(base, B), :], rows_v, sem_r)

        def wait_gather(b, idx_v, rows_v, sem_i, sem_r):
            base = b * B
            pltpu.make_async_copy(ids_hbm.at[pl.ds(base, B)], idx_v, sem_i).wait()
            pltpu.make_async_copy(
                feat_hbm.at[pl.ds(base, B), :], rows_v, sem_r).wait()

        @pl.when(wid < FULL_BLOCKS)
        def _():
            start_gather(wid, idx0_v, rows0_v, sem_gi0, sem_gr0)

        def block_body(j, carry):
            b0 = wid + (2 * j) * NW
            b1 = b0 + NW
            b0n = b0 + 2 * NW

            @pl.when(b0 < FULL_BLOCKS)
            def _():
                wait_gather(b0, idx0_v, rows0_v, sem_gi0, sem_gr0)
                pltpu.async_copy(rows0_v, sum_sh.at[idx0_v], sem_s0a, add=True)
                pltpu.async_copy(ones_v, cnt_sh.at[idx0_v], sem_s0b, add=True)

            @pl.when(b1 < FULL_BLOCKS)
            def _():
                start_gather(b1, idx1_v, rows1_v, sem_gi1, sem_gr1)

            @pl.when(b0 < FULL_BLOCKS)
            def _():
                pltpu.make_async_copy(
                    rows0_v, sum_sh.at[idx0_v], sem_s0a).wait()
                pltpu.make_async_copy(
                    ones_v, cnt_sh.at[idx0_v], sem_s0b).wait()

            @pl.when(b1 < FULL_BLOCKS)
            def _():
                wait_gather(b1, idx1_v, rows1_v, sem_gi1, sem_gr1)
                pltpu.async_copy(rows1_v, sum_sh.at[idx1_v], sem_s1a, add=True)
                pltpu.async_copy(ones_v, cnt_sh.at[idx1_v], sem_s1b, add=True)

            @pl.when(b0n < FULL_BLOCKS)
            def _():
                start_gather(b0n, idx0_v, rows0_v, sem_gi0, sem_gr0)

            @pl.when(b1 < FULL_BLOCKS)
            def _():
                pltpu.make_async_copy(
                    rows1_v, sum_sh.at[idx1_v], sem_s1a).wait()
                pltpu.make_async_copy(
                    ones_v, cnt_sh.at[idx1_v], sem_s1b).wait()

            return carry

        lax.fori_loop(0, (ITERS + 1) // 2, block_body, 0)

        @pl.when(wid == NW - 1)
        def _():
            base = FULL_BLOCKS * B
            pltpu.sync_copy(ids_hbm.at[pl.ds(base, TAIL)], idx_t)
            pltpu.sync_copy(feat_hbm.at[pl.ds(base, TAIL), :], rows_t)
            pltpu.sync_copy(ones_hbm.at[pl.ds(0, TAIL), :], ones_t)
            d1 = pltpu.async_copy(rows_t, sum_sh.at[idx_t], sem_s0a, add=True)
            d2 = pltpu.async_copy(ones_t, cnt_sh.at[idx_t], sem_s0b, add=True)
            d1.wait()
            d2.wait()

        plsc.subcore_barrier()

        seg0 = s * SEG_PER_TILE
        pltpu.sync_copy(sum_sh.at[pl.ds(seg0, SEG_PER_TILE)], zer_v)
        pltpu.sync_copy(cnt_sh.at[pl.ds(seg0, SEG_PER_TILE)], zcnt_v)
        pltpu.sync_copy(zer_v, psum_hbm.at[c, pl.ds(seg0, SEG_PER_TILE), :])
        pltpu.sync_copy(zcnt_v, pcnt_hbm.at[c, pl.ds(seg0, SEG_PER_TILE), :])

    return k(node_feat, seg_ids, ones_rows, zsum, zcnt)


def _combine_body(psum_ref, pcnt_ref, out_ref):
    sums = psum_ref[0] + psum_ref[1]
    cnts = pcnt_ref[0, :, 0:1] + pcnt_ref[1, :, 0:1]
    out_ref[...] = sums / jnp.maximum(cnts, 1.0)


def kernel(node_feat, segment_ids):
    ids32 = segment_ids.astype(jnp.int32)
    ones_rows = jnp.ones((B, CW), jnp.float32)
    zsum = jnp.zeros((SEG_PER_TILE, D), jnp.float32)
    zcnt = jnp.zeros((SEG_PER_TILE, CW), jnp.float32)
    psum, pcnt = _sc_partials(node_feat, ids32, ones_rows, zsum, zcnt)
    return pl.pallas_call(
        _combine_body,
        out_shape=jax.ShapeDtypeStruct((NSEG, D), jnp.float32),
    )(psum, pcnt)


# speedup vs baseline: 7.7641x; 1.2238x over previous
import functools

import jax
import jax.numpy as jnp
from jax import lax
from jax.experimental import pallas as pl
from jax.experimental.pallas import tpu as pltpu
from jax.experimental.pallas import tpu_sc as plsc

N_ROWS = 100000
D = 128
NSEG = 1024
NC = 2
NS = 16
NW = NC * NS
B = 128
L = 16
FULL_BLOCKS = N_ROWS // B
TAIL = N_ROWS - FULL_BLOCKS * B
ITERS = (FULL_BLOCKS + NW - 1) // NW
SEG_PER_TILE = NSEG // NS
HR = NSEG // D
HROWS = 16


def _sc_partials(node_feat, seg_ids, zsum):
    mesh = plsc.VectorSubcoreMesh(core_axis_name="c", subcore_axis_name="s",
                                  num_cores=NC, num_subcores=NS)

    @functools.partial(
        pl.kernel,
        out_type=(
            jax.ShapeDtypeStruct((NC, NSEG, D), jnp.float32),
            jax.ShapeDtypeStruct((NC, HROWS, D), jnp.float32),
        ),
        mesh=mesh,
        compiler_params=pltpu.CompilerParams(needs_layout_passes=False),
        scratch_types=[
            pltpu.VMEM((B,), jnp.int32),
            pltpu.VMEM((B,), jnp.int32),
            pltpu.VMEM((B, D), jnp.float32),
            pltpu.VMEM((B, D), jnp.float32),
            pltpu.VMEM((SEG_PER_TILE, D), jnp.float32),
            pltpu.VMEM((TAIL,), jnp.int32),
            pltpu.VMEM((TAIL, D), jnp.float32),
            pltpu.VMEM((HROWS, D), jnp.float32),
            pltpu.VMEM((L,), jnp.int32),
            pltpu.VMEM_SHARED((NSEG, D), jnp.float32),
            pltpu.VMEM_SHARED((HROWS, D), jnp.float32),
            [pltpu.SemaphoreType.DMA] * 7,
        ],
    )
    def k(feat_hbm, ids_hbm, zsum_hbm,
          psum_hbm, pcnt_hbm,
          idx0_v, idx1_v, rows0_v, rows1_v, zer_v,
          idx_t, rows_t, hist, idx_m,
          sum_sh, cnt_sh, sems):
        (sem_gi0, sem_gr0, sem_gi1, sem_gr1,
         sem_s0, sem_s1, sem_m) = sems
        c = lax.axis_index("c")
        s = lax.axis_index("s")
        wid = c * NS + s

        zvec = jnp.zeros((L,), jnp.float32)

        def zero_hist(r, carry):
            for kk in range(D // L):
                hist[r, pl.ds(kk * L, L)] = zvec
            return carry

        lax.fori_loop(0, HROWS, zero_hist, 0)
        pltpu.sync_copy(zsum_hbm, zer_v)
        pltpu.sync_copy(zer_v, sum_sh.at[pl.ds(s * SEG_PER_TILE, SEG_PER_TILE)])

        @pl.when(s == 0)
        def _():
            pltpu.sync_copy(hist, cnt_sh)

        idx_m[...] = lax.iota(jnp.int32, L)
        plsc.subcore_barrier()

        ones16 = jnp.ones((L,), jnp.float32)

        def count_block(idx_v, nvecs):
            for kk in range(nvecs):
                v = idx_v[pl.ds(kk * L, L)]
                row = lax.shift_right_logical(v, 7)
                col = lax.bitwise_and(v, 127)
                plsc.addupdate_scatter(hist, [row, col], ones16)

        def start_gather(b, idx_v, rows_v, sem_i, sem_r):
            base = b * B
            pltpu.async_copy(ids_hbm.at[pl.ds(base, B)], idx_v, sem_i)
            pltpu.async_copy(feat_hbm.at[pl.ds(base, B), :], rows_v, sem_r)

        def wait_gather(b, idx_v, rows_v, sem_i, sem_r):
            base = b * B
            pltpu.make_async_copy(ids_hbm.at[pl.ds(base, B)], idx_v, sem_i).wait()
            pltpu.make_async_copy(
                feat_hbm.at[pl.ds(base, B), :], rows_v, sem_r).wait()

        @pl.when(wid < FULL_BLOCKS)
        def _():
            start_gather(wid, idx0_v, rows0_v, sem_gi0, sem_gr0)

        def block_body(j, carry):
            b0 = wid + (2 * j) * NW
            b1 = b0 + NW
            b0n = b0 + 2 * NW

            @pl.when(b0 < FULL_BLOCKS)
            def _():
                wait_gather(b0, idx0_v, rows0_v, sem_gi0, sem_gr0)
                pltpu.async_copy(rows0_v, sum_sh.at[idx0_v], sem_s0, add=True)

            @pl.when(b1 < FULL_BLOCKS)
            def _():
                start_gather(b1, idx1_v, rows1_v, sem_gi1, sem_gr1)

            @pl.when(b0 < FULL_BLOCKS)
            def _():
                count_block(idx0_v, B // L)
                pltpu.make_async_copy(
                    rows0_v, sum_sh.at[idx0_v], sem_s0).wait()

            @pl.when(b1 < FULL_BLOCKS)
            def _():
                wait_gather(b1, idx1_v, rows1_v, sem_gi1, sem_gr1)
                pltpu.async_copy(rows1_v, sum_sh.at[idx1_v], sem_s1, add=True)

            @pl.when(b0n < FULL_BLOCKS)
            def _():
                start_gather(b0n, idx0_v, rows0_v, sem_gi0, sem_gr0)

            @pl.when(b1 < FULL_BLOCKS)
            def _():
                count_block(idx1_v, B // L)
                pltpu.make_async_copy(
                    rows1_v, sum_sh.at[idx1_v], sem_s1).wait()

            return carry

        lax.fori_loop(0, (ITERS + 1) // 2, block_body, 0)

        @pl.when(wid == NW - 1)
        def _():
            base = FULL_BLOCKS * B
            pltpu.sync_copy(ids_hbm.at[pl.ds(base, TAIL)], idx_t)
            pltpu.sync_copy(feat_hbm.at[pl.ds(base, TAIL), :], rows_t)
            d1 = pltpu.async_copy(rows_t, sum_sh.at[idx_t], sem_s0, add=True)
            count_block(idx_t, TAIL // L)
            d1.wait()

        pltpu.async_copy(hist, cnt_sh.at[idx_m], sem_m, add=True).wait()

        plsc.subcore_barrier()

        seg0 = s * SEG_PER_TILE
        pltpu.sync_copy(sum_sh.at[pl.ds(seg0, SEG_PER_TILE)], zer_v)
        pltpu.sync_copy(zer_v, psum_hbm.at[c, pl.ds(seg0, SEG_PER_TILE), :])

        @pl.when(s == 0)
        def _():
            pltpu.sync_copy(cnt_sh, hist)
            pltpu.sync_copy(hist, pcnt_hbm.at[c])

    return k(node_feat, seg_ids, zsum)


def _combine_body(psum_ref, pcnt_ref, out_ref):
    sums = psum_ref[0] + psum_ref[1]
    cnts = pcnt_ref[0] + pcnt_ref[1]
    out_ref[...] = sums / jnp.maximum(cnts, 1.0)


def kernel(node_feat, segment_ids):
    ids32 = segment_ids.astype(jnp.int32)
    zsum = jnp.zeros((SEG_PER_TILE, D), jnp.float32)
    psum, pcnt = _sc_partials(node_feat, ids32, zsum)
    pcnt_col = pcnt.reshape(NC, HROWS * D)[:, :NSEG, None]
    return pl.pallas_call(
        _combine_body,
        out_shape=jax.ShapeDtypeStruct((NSEG, D), jnp.float32),
    )(psum, pcnt_col)


# speedup vs baseline: 8.5487x; 1.1011x over previous
import functools

import jax
import jax.numpy as jnp
from jax import lax
from jax.experimental import pallas as pl
from jax.experimental.pallas import tpu as pltpu
from jax.experimental.pallas import tpu_sc as plsc

N_ROWS = 100000
D = 128
NSEG = 1024
NC = 2
NS = 16
NW = NC * NS
B = 128
L = 16
NBUF = 4
FULL_BLOCKS = N_ROWS // B
TAIL = N_ROWS - FULL_BLOCKS * B
ITERS = (FULL_BLOCKS + NW - 1) // NW
SEG_PER_TILE = NSEG // NS
HROWS = 16


def _sc_partials(node_feat, seg_ids, zsum):
    mesh = plsc.VectorSubcoreMesh(core_axis_name="c", subcore_axis_name="s",
                                  num_cores=NC, num_subcores=NS)

    @functools.partial(
        pl.kernel,
        out_type=(
            jax.ShapeDtypeStruct((NC, NSEG, D), jnp.float32),
            jax.ShapeDtypeStruct((NC, HROWS, D), jnp.float32),
        ),
        mesh=mesh,
        compiler_params=pltpu.CompilerParams(needs_layout_passes=False),
        scratch_types=[
            [pltpu.VMEM((B,), jnp.int32)] * NBUF,
            [pltpu.VMEM((B, D), jnp.float32)] * NBUF,
            pltpu.VMEM((SEG_PER_TILE, D), jnp.float32),
            pltpu.VMEM((TAIL,), jnp.int32),
            pltpu.VMEM((TAIL, D), jnp.float32),
            pltpu.VMEM((HROWS, D), jnp.float32),
            pltpu.VMEM((L,), jnp.int32),
            pltpu.VMEM_SHARED((NSEG, D), jnp.float32),
            pltpu.VMEM_SHARED((HROWS, D), jnp.float32),
            [pltpu.SemaphoreType.DMA] * NBUF,
            [pltpu.SemaphoreType.DMA] * NBUF,
            [pltpu.SemaphoreType.DMA] * NBUF,
            pltpu.SemaphoreType.DMA,
        ],
    )
    def k(feat_hbm, ids_hbm, zsum_hbm,
          psum_hbm, pcnt_hbm,
          idx_ring, rows_ring, zer_v, idx_t, rows_t, hist, idx_m,
          sum_sh, cnt_sh, sems_gi, sems_gr, sems_s, sem_m):
        c = lax.axis_index("c")
        s = lax.axis_index("s")
        wid = c * NS + s

        zvec = jnp.zeros((L,), jnp.float32)

        def zero_hist(r, carry):
            for kk in range(D // L):
                hist[r, pl.ds(kk * L, L)] = zvec
            return carry

        lax.fori_loop(0, HROWS, zero_hist, 0)
        pltpu.sync_copy(zsum_hbm, zer_v)
        pltpu.sync_copy(zer_v, sum_sh.at[pl.ds(s * SEG_PER_TILE, SEG_PER_TILE)])

        @pl.when(s == 0)
        def _():
            pltpu.sync_copy(hist, cnt_sh)

        idx_m[...] = lax.iota(jnp.int32, L)
        plsc.subcore_barrier()

        ones16 = jnp.ones((L,), jnp.float32)

        def count_block(idx_v, nvecs):
            for kk in range(nvecs):
                v = idx_v[pl.ds(kk * L, L)]
                row = lax.shift_right_logical(v, 7)
                col = lax.bitwise_and(v, 127)
                plsc.addupdate_scatter(hist, [row, col], ones16)

        def start_gather(b, kb):
            base = b * B
            pltpu.async_copy(ids_hbm.at[pl.ds(base, B)], idx_ring[kb],
                             sems_gi[kb])
            pltpu.async_copy(feat_hbm.at[pl.ds(base, B), :], rows_ring[kb],
                             sems_gr[kb])

        def wait_gather(b, kb):
            base = b * B
            pltpu.make_async_copy(
                ids_hbm.at[pl.ds(base, B)], idx_ring[kb], sems_gi[kb]).wait()
            pltpu.make_async_copy(
                feat_hbm.at[pl.ds(base, B), :], rows_ring[kb],
                sems_gr[kb]).wait()

        for kb in range(NBUF):
            b = wid + kb * NW

            @pl.when(b < FULL_BLOCKS)
            def _(b=b, kb=kb):
                start_gather(b, kb)

        def block_body(j, carry):
            for kb in range(NBUF):
                b = wid + (NBUF * j + kb) * NW
                bn = b + NBUF * NW

                @pl.when(b < FULL_BLOCKS)
                def _(b=b, kb=kb, bn=bn):
                    wait_gather(b, kb)
                    d = pltpu.async_copy(
                        rows_ring[kb], sum_sh.at[idx_ring[kb]], sems_s[kb],
                        add=True)
                    count_block(idx_ring[kb], B // L)
                    d.wait()

                    @pl.when(bn < FULL_BLOCKS)
                    def _():
                        start_gather(bn, kb)

            return carry

        lax.fori_loop(0, (ITERS + NBUF - 1) // NBUF, block_body, 0)

        @pl.when(wid == NW - 1)
        def _():
            base = FULL_BLOCKS * B
            pltpu.sync_copy(ids_hbm.at[pl.ds(base, TAIL)], idx_t)
            pltpu.sync_copy(feat_hbm.at[pl.ds(base, TAIL), :], rows_t)
            d1 = pltpu.async_copy(rows_t, sum_sh.at[idx_t], sems_s[0],
                                  add=True)
            count_block(idx_t, TAIL // L)
            d1.wait()

        pltpu.async_copy(hist, cnt_sh.at[idx_m], sem_m, add=True).wait()

        plsc.subcore_barrier()

        seg0 = s * SEG_PER_TILE
        pltpu.sync_copy(sum_sh.at[pl.ds(seg0, SEG_PER_TILE)], zer_v)
        pltpu.sync_copy(zer_v, psum_hbm.at[c, pl.ds(seg0, SEG_PER_TILE), :])

        @pl.when(s == 0)
        def _():
            pltpu.sync_copy(cnt_sh, hist)
            pltpu.sync_copy(hist, pcnt_hbm.at[c])

    return k(node_feat, seg_ids, zsum)


def _combine_body(psum_ref, pcnt_ref, out_ref):
    sums = psum_ref[0] + psum_ref[1]
    cnts = pcnt_ref[0] + pcnt_ref[1]
    out_ref[...] = sums / jnp.maximum(cnts, 1.0)


def kernel(node_feat, segment_ids):
    ids32 = segment_ids.astype(jnp.int32)
    zsum = jnp.zeros((SEG_PER_TILE, D), jnp.float32)
    psum, pcnt = _sc_partials(node_feat, ids32, zsum)
    pcnt_col = pcnt.reshape(NC, HROWS * D)[:, :NSEG, None]
    return pl.pallas_call(
        _combine_body,
        out_shape=jax.ShapeDtypeStruct((NSEG, D), jnp.float32),
    )(psum, pcnt_col)
